# baseline (speedup 1.0000x reference)
import functools

import jax
import jax.numpy as jnp
from jax import lax
from jax.experimental import pallas as pl
from jax.experimental.pallas import tpu as pltpu
from jax.experimental.pallas import tpu_sc as plsc

NUM_CORES = 2
NUM_SUBCORES = 16
NW = NUM_CORES * NUM_SUBCORES
SPAD = 56
GRP = 8


def _emb_body(n_sent, s, d, idx_hbm, table_hbm, out_hbm,
              idx_v, rows_v, gsem0, gsem1, osem0, osem1):
    wid = lax.axis_index("s") * NUM_CORES + lax.axis_index("c")
    base = wid * n_sent
    n_grp = n_sent // GRP
    gsems = (gsem0, gsem1)
    osems = (osem0, osem1)

    pltpu.sync_copy(idx_hbm.at[wid], idx_v)

    def gather_descs(gg, p):
        return [
            pltpu.make_async_copy(
                table_hbm.at[idx_v.at[gg * GRP + i].at[pl.ds(0, s)]],
                rows_v.at[p].at[i].at[pl.ds(0, s)],
                gsems[p],
            )
            for i in range(GRP)
        ]

    def wb_desc(gg, p):
        return pltpu.make_async_copy(
            rows_v.at[p].at[pl.ds(0, GRP), pl.ds(0, SPAD), pl.ds(0, d)],
            out_hbm.at[pl.ds(base + gg * GRP, GRP), pl.ds(0, SPAD), pl.ds(0, d)],
            osems[p],
        )

    for dsc in gather_descs(0, 0):
        dsc.start()

    @pl.loop(0, n_grp, step=2)
    def _group(g):
        for p in range(2):
            gg = g + p
            for dsc in gather_descs(gg, p):
                dsc.wait()
            wb_desc(gg, p).start()

            @pl.when(gg >= 1)
            def _wait_prev_wb():
                wb_desc(gg - 1, 1 - p).wait()

            @pl.when(gg + 1 < n_grp)
            def _fire_next():
                for dsc in gather_descs(gg + 1, 1 - p):
                    dsc.start()

    wb_desc(n_grp - 1, (n_grp - 1) % 2).wait()


def kernel(token_ids, table):
    b, s = token_ids.shape
    v, d = table.shape
    assert b % NW == 0 and s <= SPAD and 2 * d == 128
    n_sent = b // NW
    n_grp = n_sent // GRP
    assert n_sent % GRP == 0 and n_grp % 2 == 0

    idx = jnp.pad(token_ids.reshape(NW, n_sent, s).astype(jnp.int32),
                  ((0, 0), (0, 0), (0, 64 - s)))
    tabled = jnp.concatenate([table, table], axis=1)

    mesh = plsc.VectorSubcoreMesh(core_axis_name="c", subcore_axis_name="s")
    emb = functools.partial(
        pl.kernel,
        out_type=jax.ShapeDtypeStruct((b, SPAD, 128), jnp.float32),
        mesh=mesh,
        scratch_types=[
            pltpu.VMEM((n_sent, 64), jnp.int32),
            pltpu.VMEM((2, GRP, SPAD, 128), jnp.float32),
            pltpu.SemaphoreType.DMA,
            pltpu.SemaphoreType.DMA,
            pltpu.SemaphoreType.DMA,
            pltpu.SemaphoreType.DMA,
        ],
        compiler_params=pltpu.CompilerParams(use_tc_tiling_on_sc=False),
    )(functools.partial(_emb_body, n_sent, s, d))

    out = emb(idx, tabled)
    return out[:, :s, :d]


# speedup vs baseline: 1.3147x; 1.3147x over previous
import functools

import jax
import jax.numpy as jnp
from jax import lax
from jax.experimental import pallas as pl
from jax.experimental.pallas import tpu as pltpu
from jax.experimental.pallas import tpu_sc as plsc

NUM_CORES = 2
NUM_SUBCORES = 16
NW = NUM_CORES * NUM_SUBCORES
SPAD = 56
GRP = 16


def _emb_body(n_sent, s, d, idx_hbm, table_hbm, out_hbm,
              idx_v, rows_v, gsem0, gsem1, osem0, osem1):
    wid = lax.axis_index("s") * NUM_CORES + lax.axis_index("c")
    base = wid * n_sent
    n_grp = n_sent // GRP
    gsems = (gsem0, gsem1)
    osems = (osem0, osem1)

    pltpu.sync_copy(idx_hbm.at[wid], idx_v)

    def gather_descs(gg, p):
        return [
            pltpu.make_async_copy(
                table_hbm.at[idx_v.at[gg * GRP + i].at[pl.ds(0, s)]],
                rows_v.at[p].at[i].at[pl.ds(0, s)],
                gsems[p],
            )
            for i in range(GRP)
        ]

    def wb_desc(gg, p):
        return pltpu.make_async_copy(
            rows_v.at[p],
            out_hbm.at[pl.ds(base + gg * GRP, GRP), pl.ds(0, SPAD), pl.ds(0, d)],
            osems[p],
        )

    for dsc in gather_descs(0, 0):
        dsc.start()

    @pl.loop(0, n_grp, step=2)
    def _group(g):
        for p in range(2):
            gg = g + p
            for dsc in gather_descs(gg, p):
                dsc.wait()
            wb_desc(gg, p).start()

            @pl.when(gg >= 1)
            def _wait_prev_wb():
                wb_desc(gg - 1, 1 - p).wait()

            @pl.when(gg + 1 < n_grp)
            def _fire_next():
                for dsc in gather_descs(gg + 1, 1 - p):
                    dsc.start()

    wb_desc(n_grp - 1, (n_grp - 1) % 2).wait()


def kernel(token_ids, table):
    b, s = token_ids.shape
    v, d = table.shape
    assert b % NW == 0 and s <= SPAD and 2 * d == 128
    n_sent = b // NW
    n_grp = n_sent // GRP
    assert n_sent % GRP == 0 and n_grp % 2 == 0

    idx = jnp.pad(token_ids.reshape(NW, n_sent, s).astype(jnp.int32),
                  ((0, 0), (0, 0), (0, 64 - s)))

    mesh = plsc.VectorSubcoreMesh(core_axis_name="c", subcore_axis_name="s")
    emb = functools.partial(
        pl.kernel,
        out_type=jax.ShapeDtypeStruct((b, SPAD, 128), jnp.float32),
        mesh=mesh,
        scratch_types=[
            pltpu.VMEM((n_sent, 64), jnp.int32),
            pltpu.VMEM((2, GRP, SPAD, d), jnp.float32),
            pltpu.SemaphoreType.DMA,
            pltpu.SemaphoreType.DMA,
            pltpu.SemaphoreType.DMA,
            pltpu.SemaphoreType.DMA,
        ],
        compiler_params=pltpu.CompilerParams(use_tc_tiling_on_sc=False),
    )(functools.partial(_emb_body, n_sent, s, d))

    out = emb(idx, table)
    return out[:, :s, :d]
